# baseline (speedup 1.0000x reference)
import functools

import jax
import jax.numpy as jnp
from jax import lax
from jax.experimental import pallas as pl
from jax.experimental.pallas import tpu as pltpu
from jax.experimental.pallas import tpu_sc as plsc

N = 10000
E = 160000
D_IN = 128
D_H = 64
D_Z = 16
D_C = 2 * D_Z
LW = 128

NC = 2
NS = 16
NW = NC * NS

G = 128
EG = E // G
GPT = EG // NW
XTRA = EG - GPT * NW
GMAX = GPT + 1

NACC = 10240
RPT = NACC // NS
NDEG = 8

BM = 2048
BD = 2560
BN = 2560

NBUF = 4


def _load_groups(ei3, idx_v, w):
  pltpu.sync_copy(ei3.at[pl.ds(w * GPT, GPT)], idx_v.at[pl.ds(0, GPT)])

  @pl.when(w < XTRA)
  def _():
    pltpu.sync_copy(ei3.at[pl.ds(NW * GPT + w, 1)], idx_v.at[pl.ds(GPT, 1)])


def _deg_body(ei3, zeros1, ones1, out, idx_v, ones_v, acc, sem):
  c = lax.axis_index("c")
  s = lax.axis_index("s")
  w = c * NS + s
  ng = GPT + (w < XTRA).astype(jnp.int32)
  _load_groups(ei3, idx_v, w)
  pltpu.sync_copy(ones1, ones_v)
  pltpu.sync_copy(zeros1.at[pl.ds(s * RPT, RPT)], acc.at[pl.ds(s * RPT, RPT)])
  plsc.subcore_barrier()

  def body(g, _):
    pltpu.async_copy(ones_v, acc.at[idx_v.at[g, 1]], add=True, sem=sem)
    return ()

  lax.fori_loop(0, ng, body, (), unroll=False)

  def drain(g, _):
    pltpu.make_async_copy(ones_v, acc.at[idx_v.at[0, 1]], sem).wait()
    return ()

  lax.fori_loop(0, ng, drain, (), unroll=False)
  plsc.subcore_barrier()
  pltpu.sync_copy(acc.at[pl.ds(s * RPT, RPT)], out.at[c].at[pl.ds(s * RPT, RPT)])


def _make_deg_call():
  mesh = plsc.VectorSubcoreMesh(
      core_axis_name="c", subcore_axis_name="s", num_cores=NC, num_subcores=NS)
  return pl.kernel(
      _deg_body,
      out_type=jax.ShapeDtypeStruct((NDEG, NACC), jnp.float32),
      mesh=mesh,
      compiler_params=pltpu.CompilerParams(use_tc_tiling_on_sc=False),
      scratch_types=[
          pltpu.VMEM((GMAX, 2, G), jnp.int32),
          pltpu.VMEM((G,), jnp.float32),
          pltpu.VMEM_SHARED((NACC,), jnp.float32),
          pltpu.SemaphoreType.DMA,
      ],
  )


def _agg_body(d, ei3, hs, zeros2, out, idx_v, rows, gsems, ssems, acc):
  c = lax.axis_index("c")
  s = lax.axis_index("s")
  w = c * NS + s
  ng = GPT + (w < XTRA).astype(jnp.int32)
  _load_groups(ei3, idx_v, w)
  pltpu.sync_copy(zeros2.at[pl.ds(s * RPT, RPT)], acc.at[pl.ds(s * RPT, RPT)])
  plsc.subcore_barrier()

  for k in range(NBUF):

    @pl.when(k < ng)
    def _(k=k):
      pltpu.async_copy(hs.at[idx_v.at[k, 0]], rows.at[k], gsems.at[k])

  def rnd(r, _):
    g0 = NBUF * r
    for k in range(NBUF):
      g = g0 + k

      @pl.when(g < ng)
      def _(k=k, g=g):
        pltpu.make_async_copy(hs.at[idx_v.at[g, 0]], rows.at[k],
                              gsems.at[k]).wait()
        pltpu.async_copy(rows.at[k], acc.at[idx_v.at[g, 1]], add=True,
                         sem=ssems.at[k])

        @pl.when(g + NBUF < ng)
        def _():
          pltpu.make_async_copy(rows.at[k], acc.at[idx_v.at[0, 1]],
                                ssems.at[k]).wait()
          pltpu.async_copy(hs.at[idx_v.at[g + NBUF, 0]], rows.at[k],
                           gsems.at[k])

    return ()

  lax.fori_loop(0, (GMAX + NBUF - 1) // NBUF, rnd, (), unroll=False)

  for k in range(NBUF):

    @pl.when(k < ng)
    def _(k=k):
      pltpu.make_async_copy(rows.at[k], acc.at[idx_v.at[0, 1]],
                            ssems.at[k]).wait()

  plsc.subcore_barrier()
  pltpu.sync_copy(acc.at[pl.ds(s * RPT, RPT)],
                  out.at[c].at[pl.ds(s * RPT, RPT), pl.ds(0, d)])


def _make_agg_call(d):
  mesh = plsc.VectorSubcoreMesh(
      core_axis_name="c", subcore_axis_name="s", num_cores=NC, num_subcores=NS)
  return pl.kernel(
      functools.partial(_agg_body, d),
      out_type=jax.ShapeDtypeStruct((NC, NACC, LW), jnp.float32),
      mesh=mesh,
      compiler_params=pltpu.CompilerParams(use_tc_tiling_on_sc=False),
      scratch_types=[
          pltpu.VMEM((GMAX, 2, G), jnp.int32),
          pltpu.VMEM((NBUF, G, d), jnp.float32),
          pltpu.SemaphoreType.DMA((NBUF,)),
          pltpu.SemaphoreType.DMA((NBUF,)),
          pltpu.VMEM_SHARED((NACC, d), jnp.float32),
      ],
  )


def _dinv_col(deg_ref):
  degsum = deg_ref[0:1, :] + deg_ref[1:2, :] + 1.0
  return lax.transpose(lax.rsqrt(degsum), (1, 0))


def _enc1_body(x_ref, w1_ref, b1_ref, deg_ref, hs_ref):
  h = jnp.dot(x_ref[...], w1_ref[...], preferred_element_type=jnp.float32)
  h = h + b1_ref[...]
  hs_ref[...] = h * _dinv_col(deg_ref)


def _enc2_body(agg_ref, hs1_ref, deg_ref, wcat_ref, bcat_ref, hs2_ref):
  dinv = _dinv_col(deg_ref)
  a1 = dinv * (agg_ref[0, :, :D_H] + agg_ref[1, :, :D_H] + hs1_ref[...])
  h1 = jnp.maximum(a1, 0.0)
  hcat = jnp.dot(h1, wcat_ref[...], preferred_element_type=jnp.float32)
  hcat = hcat + bcat_ref[...]
  hs2_ref[...] = hcat * dinv


def _rep_body(agg_ref, hs2_ref, deg_ref, eps_ref, z_ref, mu_ref, ls_ref):
  dinv = _dinv_col(deg_ref)
  acat = dinv * (agg_ref[0, :, :D_C] + agg_ref[1, :, :D_C] + hs2_ref[...])
  mu = acat[:, :D_Z]
  ls = acat[:, D_Z:]
  mu_ref[...] = mu
  ls_ref[...] = ls
  z_ref[...] = mu + eps_ref[...] * jnp.exp(ls)


def _dec_body(zi_ref, zj_ref, adj_ref):
  adj_ref[...] = lax.dot_general(
      zi_ref[...], zj_ref[...], (((1,), (1,)), ((), ())),
      preferred_element_type=jnp.float32)


def kernel(x, edge_index, W1, b1, Wmu, bmu, Wls, bls, eps):
  ei3 = jnp.transpose(edge_index.astype(jnp.int32).reshape(2, EG, G), (1, 0, 2))

  zeros1 = jnp.zeros((NACC,), jnp.float32)
  ones1 = jnp.ones((G,), jnp.float32)
  zeros64 = jnp.zeros((NACC, D_H), jnp.float32)
  zeros32 = jnp.zeros((NACC, D_C), jnp.float32)

  wcat = jnp.concatenate([Wmu, Wls], axis=1)
  bcat = jnp.concatenate([bmu, bls]).reshape(1, D_C)
  b1r = b1.reshape(1, D_H)

  deg2 = _make_deg_call()(ei3, zeros1, ones1)

  gm = -(-N // BM)
  deg_spec = pl.BlockSpec((NDEG, BM), lambda i: (0, i))
  agg1_spec = pl.BlockSpec((NC, BM, LW), lambda i: (0, i, 0))
  agg2_spec = pl.BlockSpec((NC, BM, LW), lambda i: (0, i, 0))
  tc_params = pltpu.CompilerParams(dimension_semantics=("parallel",))

  hs1 = pl.pallas_call(
      _enc1_body,
      grid=(gm,),
      in_specs=[
          pl.BlockSpec((BM, D_IN), lambda i: (i, 0)),
          pl.BlockSpec((D_IN, D_H), lambda i: (0, 0)),
          pl.BlockSpec((1, D_H), lambda i: (0, 0)),
          deg_spec,
      ],
      out_specs=pl.BlockSpec((BM, D_H), lambda i: (i, 0)),
      out_shape=jax.ShapeDtypeStruct((N, D_H), jnp.float32),
      compiler_params=tc_params,
  )(x, W1, b1r, deg2)

  agg1 = _make_agg_call(D_H)(ei3, hs1, zeros64)

  hs2 = pl.pallas_call(
      _enc2_body,
      grid=(gm,),
      in_specs=[
          agg1_spec,
          pl.BlockSpec((BM, D_H), lambda i: (i, 0)),
          deg_spec,
          pl.BlockSpec((D_H, D_C), lambda i: (0, 0)),
          pl.BlockSpec((1, D_C), lambda i: (0, 0)),
      ],
      out_specs=pl.BlockSpec((BM, D_C), lambda i: (i, 0)),
      out_shape=jax.ShapeDtypeStruct((N, D_C), jnp.float32),
      compiler_params=tc_params,
  )(agg1, hs1, deg2, wcat, bcat)

  agg2 = _make_agg_call(D_C)(ei3, hs2, zeros32)

  z, mu, ls = pl.pallas_call(
      _rep_body,
      grid=(gm,),
      in_specs=[
          agg2_spec,
          pl.BlockSpec((BM, D_C), lambda i: (i, 0)),
          deg_spec,
          pl.BlockSpec((BM, D_Z), lambda i: (i, 0)),
      ],
      out_specs=[
          pl.BlockSpec((BM, D_Z), lambda i: (i, 0)),
          pl.BlockSpec((BM, D_Z), lambda i: (i, 0)),
          pl.BlockSpec((BM, D_Z), lambda i: (i, 0)),
      ],
      out_shape=[
          jax.ShapeDtypeStruct((N, D_Z), jnp.float32),
          jax.ShapeDtypeStruct((N, D_Z), jnp.float32),
          jax.ShapeDtypeStruct((N, D_Z), jnp.float32),
      ],
      compiler_params=tc_params,
  )(agg2, hs2, deg2, eps)

  adj = pl.pallas_call(
      _dec_body,
      grid=(-(-N // BD), -(-N // BN)),
      in_specs=[
          pl.BlockSpec((BD, D_Z), lambda i, j: (i, 0)),
          pl.BlockSpec((BN, D_Z), lambda i, j: (j, 0)),
      ],
      out_specs=pl.BlockSpec((BD, BN), lambda i, j: (i, j)),
      out_shape=jax.ShapeDtypeStruct((N, N), jnp.float32),
      compiler_params=pltpu.CompilerParams(
          dimension_semantics=("parallel", "parallel")),
  )(z, z)

  return adj, mu, ls


# speedup vs baseline: 1.0467x; 1.0467x over previous
import functools

import jax
import jax.numpy as jnp
from jax import lax
from jax.experimental import pallas as pl
from jax.experimental.pallas import tpu as pltpu
from jax.experimental.pallas import tpu_sc as plsc

N = 10000
E = 160000
D_IN = 128
D_H = 64
D_Z = 16
D_C = 2 * D_Z
LW = 128

NC = 2
NS = 16
NW = NC * NS

G = 128
EG = E // G
GPT = EG // NW
XTRA = EG - GPT * NW
GMAX = GPT + 1

NACC = 10240
RPT = NACC // NS
NDEG = 8

BM = 4096
BD = 2560
BN = 2560

NBUF = 4


def _load_groups(ei3, idx_v, w):
  pltpu.sync_copy(ei3.at[pl.ds(w * GPT, GPT)], idx_v.at[pl.ds(0, GPT)])

  @pl.when(w < XTRA)
  def _():
    pltpu.sync_copy(ei3.at[pl.ds(NW * GPT + w, 1)], idx_v.at[pl.ds(GPT, 1)])


def _deg_body(ei3, zeros1, ones1, out, idx_v, ones_v, acc, sem):
  c = lax.axis_index("c")
  s = lax.axis_index("s")
  w = c * NS + s
  ng = GPT + (w < XTRA).astype(jnp.int32)
  _load_groups(ei3, idx_v, w)
  pltpu.sync_copy(ones1, ones_v)
  pltpu.sync_copy(zeros1.at[pl.ds(s * RPT, RPT)], acc.at[pl.ds(s * RPT, RPT)])
  plsc.subcore_barrier()

  def body(g, _):
    pltpu.async_copy(ones_v, acc.at[idx_v.at[g, 1]], add=True, sem=sem)
    return ()

  lax.fori_loop(0, ng, body, (), unroll=False)

  def drain(g, _):
    pltpu.make_async_copy(ones_v, acc.at[idx_v.at[0, 1]], sem).wait()
    return ()

  lax.fori_loop(0, ng, drain, (), unroll=False)
  plsc.subcore_barrier()
  pltpu.sync_copy(acc.at[pl.ds(s * RPT, RPT)], out.at[c].at[pl.ds(s * RPT, RPT)])


def _make_deg_call():
  mesh = plsc.VectorSubcoreMesh(
      core_axis_name="c", subcore_axis_name="s", num_cores=NC, num_subcores=NS)
  return pl.kernel(
      _deg_body,
      out_type=jax.ShapeDtypeStruct((NDEG, NACC), jnp.float32),
      mesh=mesh,
      compiler_params=pltpu.CompilerParams(use_tc_tiling_on_sc=False),
      scratch_types=[
          pltpu.VMEM((GMAX, 2, G), jnp.int32),
          pltpu.VMEM((G,), jnp.float32),
          pltpu.VMEM_SHARED((NACC,), jnp.float32),
          pltpu.SemaphoreType.DMA,
      ],
  )


def _agg_body(d, ei3, hs, zeros2, out, idx_v, rows, gsems, ssems, acc):
  c = lax.axis_index("c")
  s = lax.axis_index("s")
  w = c * NS + s
  ng = GPT + (w < XTRA).astype(jnp.int32)
  _load_groups(ei3, idx_v, w)
  pltpu.sync_copy(zeros2.at[pl.ds(s * RPT, RPT)], acc.at[pl.ds(s * RPT, RPT)])
  plsc.subcore_barrier()

  for k in range(NBUF):

    @pl.when(k < ng)
    def _(k=k):
      pltpu.async_copy(hs.at[idx_v.at[k, 0]], rows.at[k], gsems.at[k])

  def rnd(r, _):
    g0 = NBUF * r
    for k in range(NBUF):
      g = g0 + k

      @pl.when(g < ng)
      def _(k=k, g=g):
        pltpu.make_async_copy(hs.at[idx_v.at[g, 0]], rows.at[k],
                              gsems.at[k]).wait()
        pltpu.async_copy(rows.at[k], acc.at[idx_v.at[g, 1]], add=True,
                         sem=ssems.at[k])

        @pl.when(g + NBUF < ng)
        def _():
          pltpu.make_async_copy(rows.at[k], acc.at[idx_v.at[0, 1]],
                                ssems.at[k]).wait()
          pltpu.async_copy(hs.at[idx_v.at[g + NBUF, 0]], rows.at[k],
                           gsems.at[k])

    return ()

  lax.fori_loop(0, (GMAX + NBUF - 1) // NBUF, rnd, (), unroll=False)

  for k in range(NBUF):

    @pl.when(k < ng)
    def _(k=k):
      pltpu.make_async_copy(rows.at[k], acc.at[idx_v.at[0, 1]],
                            ssems.at[k]).wait()

  plsc.subcore_barrier()
  pltpu.sync_copy(acc.at[pl.ds(s * RPT, RPT)],
                  out.at[c].at[pl.ds(s * RPT, RPT), pl.ds(0, d)])


def _make_agg_call(d):
  mesh = plsc.VectorSubcoreMesh(
      core_axis_name="c", subcore_axis_name="s", num_cores=NC, num_subcores=NS)
  return pl.kernel(
      functools.partial(_agg_body, d),
      out_type=jax.ShapeDtypeStruct((NC, NACC, LW), jnp.float32),
      mesh=mesh,
      compiler_params=pltpu.CompilerParams(use_tc_tiling_on_sc=False),
      scratch_types=[
          pltpu.VMEM((GMAX, 2, G), jnp.int32),
          pltpu.VMEM((NBUF, G, d), jnp.float32),
          pltpu.SemaphoreType.DMA((NBUF,)),
          pltpu.SemaphoreType.DMA((NBUF,)),
          pltpu.VMEM_SHARED((NACC, d), jnp.float32),
      ],
  )


def _dinv_col(deg_ref):
  degsum = deg_ref[0:1, :] + deg_ref[1:2, :] + 1.0
  return lax.transpose(lax.rsqrt(degsum), (1, 0))


def _enc1_body(x_ref, w1_ref, b1_ref, deg_ref, hs_ref):
  h = jnp.dot(x_ref[...], w1_ref[...], preferred_element_type=jnp.float32)
  h = h + b1_ref[...]
  hs_ref[...] = h * _dinv_col(deg_ref)


def _enc2_body(agg_ref, hs1_ref, deg_ref, wcat_ref, bcat_ref, hs2_ref):
  dinv = _dinv_col(deg_ref)
  a1 = dinv * (agg_ref[0, :, :D_H] + agg_ref[1, :, :D_H] + hs1_ref[...])
  h1 = jnp.maximum(a1, 0.0)
  hcat = jnp.dot(h1, wcat_ref[...], preferred_element_type=jnp.float32)
  hcat = hcat + bcat_ref[...]
  hs2_ref[...] = hcat * dinv


def _rep_body(agg_ref, hs2_ref, deg_ref, epst_ref, z_ref, mut_ref, lst_ref):
  dinv = _dinv_col(deg_ref)
  acat = dinv * (agg_ref[0, :, :D_C] + agg_ref[1, :, :D_C] + hs2_ref[...])
  mu = acat[:, :D_Z]
  ls = acat[:, D_Z:]
  mut_ref[...] = lax.transpose(mu, (1, 0))
  lst_ref[...] = lax.transpose(ls, (1, 0))
  eps = lax.transpose(epst_ref[...], (1, 0))
  z_ref[...] = mu + eps * jnp.exp(ls)


def _dec_body(zi_ref, zj_ref, adj_ref):
  adj_ref[...] = lax.dot_general(
      zi_ref[...], zj_ref[...], (((1,), (1,)), ((), ())),
      preferred_element_type=jnp.float32)


def kernel(x, edge_index, W1, b1, Wmu, bmu, Wls, bls, eps):
  ei3 = jnp.transpose(edge_index.astype(jnp.int32).reshape(2, EG, G), (1, 0, 2))

  zeros1 = jnp.zeros((NACC,), jnp.float32)
  ones1 = jnp.ones((G,), jnp.float32)
  zeros64 = jnp.zeros((NACC, D_H), jnp.float32)
  zeros32 = jnp.zeros((NACC, D_C), jnp.float32)

  wcat = jnp.concatenate([Wmu, Wls], axis=1)
  bcat = jnp.concatenate([bmu, bls]).reshape(1, D_C)
  b1r = b1.reshape(1, D_H)

  deg2 = _make_deg_call()(ei3, zeros1, ones1)

  gm = -(-N // BM)
  deg_spec = pl.BlockSpec((NDEG, BM), lambda i: (0, i))
  agg1_spec = pl.BlockSpec((NC, BM, LW), lambda i: (0, i, 0))
  agg2_spec = pl.BlockSpec((NC, BM, LW), lambda i: (0, i, 0))
  tc_params = pltpu.CompilerParams(dimension_semantics=("parallel",))

  hs1 = pl.pallas_call(
      _enc1_body,
      grid=(gm,),
      in_specs=[
          pl.BlockSpec((BM, D_IN), lambda i: (i, 0)),
          pl.BlockSpec((D_IN, D_H), lambda i: (0, 0)),
          pl.BlockSpec((1, D_H), lambda i: (0, 0)),
          deg_spec,
      ],
      out_specs=pl.BlockSpec((BM, D_H), lambda i: (i, 0)),
      out_shape=jax.ShapeDtypeStruct((N, D_H), jnp.float32),
      compiler_params=tc_params,
  )(x, W1, b1r, deg2)

  agg1 = _make_agg_call(D_H)(ei3, hs1, zeros64)

  hs2 = pl.pallas_call(
      _enc2_body,
      grid=(gm,),
      in_specs=[
          agg1_spec,
          pl.BlockSpec((BM, D_H), lambda i: (i, 0)),
          deg_spec,
          pl.BlockSpec((D_H, D_C), lambda i: (0, 0)),
          pl.BlockSpec((1, D_C), lambda i: (0, 0)),
      ],
      out_specs=pl.BlockSpec((BM, D_C), lambda i: (i, 0)),
      out_shape=jax.ShapeDtypeStruct((N, D_C), jnp.float32),
      compiler_params=tc_params,
  )(agg1, hs1, deg2, wcat, bcat)

  agg2 = _make_agg_call(D_C)(ei3, hs2, zeros32)

  epst = jnp.transpose(eps)
  z, mut, lst = pl.pallas_call(
      _rep_body,
      grid=(gm,),
      in_specs=[
          agg2_spec,
          pl.BlockSpec((BM, D_C), lambda i: (i, 0)),
          deg_spec,
          pl.BlockSpec((D_Z, BM), lambda i: (0, i)),
      ],
      out_specs=[
          pl.BlockSpec((BM, D_Z), lambda i: (i, 0)),
          pl.BlockSpec((D_Z, BM), lambda i: (0, i)),
          pl.BlockSpec((D_Z, BM), lambda i: (0, i)),
      ],
      out_shape=[
          jax.ShapeDtypeStruct((N, D_Z), jnp.float32),
          jax.ShapeDtypeStruct((D_Z, N), jnp.float32),
          jax.ShapeDtypeStruct((D_Z, N), jnp.float32),
      ],
      compiler_params=tc_params,
  )(agg2, hs2, deg2, epst)
  mu = jnp.transpose(mut)
  ls = jnp.transpose(lst)

  adj = pl.pallas_call(
      _dec_body,
      grid=(-(-N // BD), -(-N // BN)),
      in_specs=[
          pl.BlockSpec((BD, D_Z), lambda i, j: (i, 0)),
          pl.BlockSpec((BN, D_Z), lambda i, j: (j, 0)),
      ],
      out_specs=pl.BlockSpec((BD, BN), lambda i, j: (i, j)),
      out_shape=jax.ShapeDtypeStruct((N, N), jnp.float32),
      compiler_params=pltpu.CompilerParams(
          dimension_semantics=("parallel", "parallel")),
  )(z, z)

  return adj, mu, ls


# speedup vs baseline: 1.0554x; 1.0084x over previous
import functools

import jax
import jax.numpy as jnp
from jax import lax
from jax.experimental import pallas as pl
from jax.experimental.pallas import tpu as pltpu
from jax.experimental.pallas import tpu_sc as plsc

N = 10000
E = 160000
D_IN = 128
D_H = 64
D_Z = 16
D_C = 2 * D_Z
LW = 128

NC = 2
NS = 16
NW = NC * NS

G = 128
EG = E // G
GPT = EG // NW
XTRA = EG - GPT * NW
GMAX = GPT + 1

NACC = 10240
RPT = NACC // NS
NDEG = 8

BM = 4096
BD = 2560
BN = 2560

NBUF = 4


def _load_groups(ei3, idx_v, w):
  pltpu.sync_copy(ei3.at[pl.ds(w * GPT, GPT)], idx_v.at[pl.ds(0, GPT)])

  @pl.when(w < XTRA)
  def _():
    pltpu.sync_copy(ei3.at[pl.ds(NW * GPT + w, 1)], idx_v.at[pl.ds(GPT, 1)])


def _deg_body(ei3, zeros1, ones1, out, idx_v, ones_v, acc, sem):
  c = lax.axis_index("c")
  s = lax.axis_index("s")
  w = c * NS + s
  ng = GPT + (w < XTRA).astype(jnp.int32)
  _load_groups(ei3, idx_v, w)
  pltpu.sync_copy(ones1, ones_v)
  pltpu.sync_copy(zeros1.at[pl.ds(s * RPT, RPT)], acc.at[pl.ds(s * RPT, RPT)])
  plsc.subcore_barrier()

  def body(g, _):
    pltpu.async_copy(ones_v, acc.at[idx_v.at[g, 1]], add=True, sem=sem)
    return ()

  lax.fori_loop(0, ng, body, (), unroll=False)

  def drain(g, _):
    pltpu.make_async_copy(ones_v, acc.at[idx_v.at[0, 1]], sem).wait()
    return ()

  lax.fori_loop(0, ng, drain, (), unroll=False)
  plsc.subcore_barrier()
  pltpu.sync_copy(acc.at[pl.ds(s * RPT, RPT)], out.at[c].at[pl.ds(s * RPT, RPT)])


def _make_deg_call():
  mesh = plsc.VectorSubcoreMesh(
      core_axis_name="c", subcore_axis_name="s", num_cores=NC, num_subcores=NS)
  return pl.kernel(
      _deg_body,
      out_type=jax.ShapeDtypeStruct((NDEG, NACC), jnp.float32),
      mesh=mesh,
      compiler_params=pltpu.CompilerParams(use_tc_tiling_on_sc=False),
      scratch_types=[
          pltpu.VMEM((GMAX, 2, G), jnp.int32),
          pltpu.VMEM((G,), jnp.float32),
          pltpu.VMEM_SHARED((NACC,), jnp.float32),
          pltpu.SemaphoreType.DMA,
      ],
  )


def _agg_body(d, ei3, hs, zeros2, out, idx_v, rows, gsems, ssems, acc):
  c = lax.axis_index("c")
  s = lax.axis_index("s")
  w = c * NS + s
  ng = GPT + (w < XTRA).astype(jnp.int32)
  _load_groups(ei3, idx_v, w)
  pltpu.sync_copy(zeros2.at[pl.ds(s * RPT, RPT)], acc.at[pl.ds(s * RPT, RPT)])
  plsc.subcore_barrier()

  for k in range(NBUF):

    @pl.when(k < ng)
    def _(k=k):
      pltpu.async_copy(hs.at[idx_v.at[k, 0]], rows.at[k], gsems.at[k])

  def rnd(r, _):
    g0 = NBUF * r
    for k in range(NBUF):
      g = g0 + k

      @pl.when(g < ng)
      def _(k=k, g=g):
        pltpu.make_async_copy(hs.at[idx_v.at[g, 0]], rows.at[k],
                              gsems.at[k]).wait()
        pltpu.async_copy(rows.at[k], acc.at[idx_v.at[g, 1]], add=True,
                         sem=ssems.at[k])

        @pl.when(g + NBUF < ng)
        def _():
          pltpu.make_async_copy(rows.at[k], acc.at[idx_v.at[0, 1]],
                                ssems.at[k]).wait()
          pltpu.async_copy(hs.at[idx_v.at[g + NBUF, 0]], rows.at[k],
                           gsems.at[k])

    return ()

  lax.fori_loop(0, (GMAX + NBUF - 1) // NBUF, rnd, (), unroll=False)

  for k in range(NBUF):

    @pl.when(k < ng)
    def _(k=k):
      pltpu.make_async_copy(rows.at[k], acc.at[idx_v.at[0, 1]],
                            ssems.at[k]).wait()

  plsc.subcore_barrier()
  pltpu.sync_copy(acc.at[pl.ds(s * RPT, RPT)],
                  out.at[c].at[pl.ds(s * RPT, RPT), pl.ds(0, d)])


def _make_agg_call(d):
  mesh = plsc.VectorSubcoreMesh(
      core_axis_name="c", subcore_axis_name="s", num_cores=NC, num_subcores=NS)
  return pl.kernel(
      functools.partial(_agg_body, d),
      out_type=jax.ShapeDtypeStruct((NC, NACC, LW), jnp.float32),
      mesh=mesh,
      compiler_params=pltpu.CompilerParams(use_tc_tiling_on_sc=False),
      scratch_types=[
          pltpu.VMEM((GMAX, 2, G), jnp.int32),
          pltpu.VMEM((NBUF, G, d), jnp.float32),
          pltpu.SemaphoreType.DMA((NBUF,)),
          pltpu.SemaphoreType.DMA((NBUF,)),
          pltpu.VMEM_SHARED((NACC, d), jnp.float32),
      ],
  )


def _dinv_col(deg_ref):
  degsum = deg_ref[0:1, :] + deg_ref[1:2, :] + 1.0
  return lax.transpose(lax.rsqrt(degsum), (1, 0))


def _enc1_body(x_ref, w1_ref, b1_ref, deg_ref, hs_ref):
  h = jnp.dot(x_ref[...], w1_ref[...], preferred_element_type=jnp.float32)
  h = h + b1_ref[...]
  hs_ref[...] = h * _dinv_col(deg_ref)


def _enc2_body(agg_ref, hs1_ref, deg_ref, wcat_ref, bcat_ref, hs2_ref):
  dinv = _dinv_col(deg_ref)
  a1 = dinv * (agg_ref[0, :, :D_H] + agg_ref[1, :, :D_H] + hs1_ref[...])
  h1 = jnp.maximum(a1, 0.0)
  hcat = jnp.dot(h1, wcat_ref[...], preferred_element_type=jnp.float32)
  hcat = hcat + bcat_ref[...]
  hs2_ref[...] = hcat * dinv


def _rep_body(agg_ref, hs2_ref, deg_ref, epst_ref, zt_ref, mut_ref, lst_ref):
  dinv = _dinv_col(deg_ref)
  acat = dinv * (agg_ref[0, :, :D_C] + agg_ref[1, :, :D_C] + hs2_ref[...])
  mu = acat[:, :D_Z]
  ls = acat[:, D_Z:]
  mut_ref[...] = lax.transpose(mu, (1, 0))
  lst_ref[...] = lax.transpose(ls, (1, 0))
  eps = lax.transpose(epst_ref[...], (1, 0))
  zt_ref[...] = lax.transpose(mu + eps * jnp.exp(ls), (1, 0))


def _dec_body(zi_ref, zj_ref, adj_ref):
  adj_ref[...] = lax.dot_general(
      zi_ref[...], zj_ref[...], (((0,), (0,)), ((), ())),
      preferred_element_type=jnp.float32)


def kernel(x, edge_index, W1, b1, Wmu, bmu, Wls, bls, eps):
  ei3 = jnp.transpose(edge_index.astype(jnp.int32).reshape(2, EG, G), (1, 0, 2))

  zeros1 = jnp.zeros((NACC,), jnp.float32)
  ones1 = jnp.ones((G,), jnp.float32)
  zeros64 = jnp.zeros((NACC, D_H), jnp.float32)
  zeros32 = jnp.zeros((NACC, D_C), jnp.float32)

  wcat = jnp.concatenate([Wmu, Wls], axis=1)
  bcat = jnp.concatenate([bmu, bls]).reshape(1, D_C)
  b1r = b1.reshape(1, D_H)

  deg2 = _make_deg_call()(ei3, zeros1, ones1)

  gm = -(-N // BM)
  deg_spec = pl.BlockSpec((NDEG, BM), lambda i: (0, i))
  agg1_spec = pl.BlockSpec((NC, BM, LW), lambda i: (0, i, 0))
  agg2_spec = pl.BlockSpec((NC, BM, LW), lambda i: (0, i, 0))
  tc_params = pltpu.CompilerParams(dimension_semantics=("parallel",))

  hs1 = pl.pallas_call(
      _enc1_body,
      grid=(gm,),
      in_specs=[
          pl.BlockSpec((BM, D_IN), lambda i: (i, 0)),
          pl.BlockSpec((D_IN, D_H), lambda i: (0, 0)),
          pl.BlockSpec((1, D_H), lambda i: (0, 0)),
          deg_spec,
      ],
      out_specs=pl.BlockSpec((BM, D_H), lambda i: (i, 0)),
      out_shape=jax.ShapeDtypeStruct((N, D_H), jnp.float32),
      compiler_params=tc_params,
  )(x, W1, b1r, deg2)

  agg1 = _make_agg_call(D_H)(ei3, hs1, zeros64)

  hs2 = pl.pallas_call(
      _enc2_body,
      grid=(gm,),
      in_specs=[
          agg1_spec,
          pl.BlockSpec((BM, D_H), lambda i: (i, 0)),
          deg_spec,
          pl.BlockSpec((D_H, D_C), lambda i: (0, 0)),
          pl.BlockSpec((1, D_C), lambda i: (0, 0)),
      ],
      out_specs=pl.BlockSpec((BM, D_C), lambda i: (i, 0)),
      out_shape=jax.ShapeDtypeStruct((N, D_C), jnp.float32),
      compiler_params=tc_params,
  )(agg1, hs1, deg2, wcat, bcat)

  agg2 = _make_agg_call(D_C)(ei3, hs2, zeros32)

  epst = jnp.transpose(eps)
  zt, mut, lst = pl.pallas_call(
      _rep_body,
      grid=(gm,),
      in_specs=[
          agg2_spec,
          pl.BlockSpec((BM, D_C), lambda i: (i, 0)),
          deg_spec,
          pl.BlockSpec((D_Z, BM), lambda i: (0, i)),
      ],
      out_specs=[
          pl.BlockSpec((D_Z, BM), lambda i: (0, i)),
          pl.BlockSpec((D_Z, BM), lambda i: (0, i)),
          pl.BlockSpec((D_Z, BM), lambda i: (0, i)),
      ],
      out_shape=[
          jax.ShapeDtypeStruct((D_Z, N), jnp.float32),
          jax.ShapeDtypeStruct((D_Z, N), jnp.float32),
          jax.ShapeDtypeStruct((D_Z, N), jnp.float32),
      ],
      compiler_params=tc_params,
  )(agg2, hs2, deg2, epst)
  mu = jnp.transpose(mut)
  ls = jnp.transpose(lst)

  adj = pl.pallas_call(
      _dec_body,
      grid=(-(-N // BD), -(-N // BN)),
      in_specs=[
          pl.BlockSpec((D_Z, BD), lambda i, j: (0, i)),
          pl.BlockSpec((D_Z, BN), lambda i, j: (0, j)),
      ],
      out_specs=pl.BlockSpec((BD, BN), lambda i, j: (i, j)),
      out_shape=jax.ShapeDtypeStruct((N, N), jnp.float32),
      compiler_params=pltpu.CompilerParams(
          dimension_semantics=("parallel", "parallel")),
  )(zt, zt)

  return adj, mu, ls
